# baseline (speedup 1.0000x reference)
import functools

import jax
import jax.numpy as jnp
from jax import lax
from jax.experimental import pallas as pl
from jax.experimental.pallas import tpu as pltpu
from jax.experimental.pallas import tpu_sc as plsc

N = 10000
E = 160000
D = 128

NC = 2
NS = 16
NW = NC * NS

WPC = 256
NWINP = 768
MW = 384

F32 = jnp.float32


def _leaky(v):
    return jnp.where(v >= 0, v, 0.01 * v)


def _node_embed_body(x_ref, w_ref, b_ref, o_ref):
    o_ref[...] = (
        jnp.dot(x_ref[...], w_ref[...], preferred_element_type=F32) + b_ref[...]
    )


def _node_embed(x, w, b, rows_blk):
    n = x.shape[0]
    grid = n // rows_blk
    return pl.pallas_call(
        _node_embed_body,
        grid=(grid,),
        in_specs=[
            pl.BlockSpec((rows_blk, x.shape[1]), lambda i: (i, 0)),
            pl.BlockSpec(w.shape, lambda i: (0, 0)),
            pl.BlockSpec((1, D), lambda i: (0, 0)),
        ],
        out_specs=pl.BlockSpec((rows_blk, D), lambda i: (i, 0)),
        out_shape=jax.ShapeDtypeStruct((n, D), F32),
    )(x, w, b.reshape(1, D))


def _feat_embed_body(d_ref, c_ref, w_ref, b_ref, o_ref, *, bins, gamma):
    d = d_ref[...]
    c = c_ref[...]
    w = w_ref[...]
    acc = jnp.zeros((d.shape[0], D), F32)
    for i in range(3):
        di = d[:, i : i + 1]
        r = jnp.exp(-gamma * (di - c) ** 2)
        acc += jnp.dot(r, w[i * bins : (i + 1) * bins, :], preferred_element_type=F32)
    acc += jnp.dot(d[:, 3:], w[3 * bins :, :], preferred_element_type=F32)
    o_ref[...] = acc + b_ref[...]


def _feat_embed(d, w, b, bins, rows_blk):
    n, f = d.shape
    gamma = 1.0 / (8.0 / bins) ** 2
    centers = jnp.linspace(0.0, 8.0, bins, dtype=F32).reshape(1, bins)
    grid = n // rows_blk
    body = functools.partial(_feat_embed_body, bins=bins, gamma=gamma)
    return pl.pallas_call(
        body,
        grid=(grid,),
        in_specs=[
            pl.BlockSpec((rows_blk, f), lambda i: (i, 0)),
            pl.BlockSpec((1, bins), lambda i: (0, 0)),
            pl.BlockSpec(w.shape, lambda i: (0, 0)),
            pl.BlockSpec((1, D), lambda i: (0, 0)),
        ],
        out_specs=pl.BlockSpec((rows_blk, D), lambda i: (i, 0)),
        out_shape=jax.ShapeDtypeStruct((n, D), F32),
    )(d, centers, w, b.reshape(1, D))


def _proj_body(h_ref, w_ref, o1_ref, o2_ref):
    t = jnp.dot(h_ref[...], w_ref[...], preferred_element_type=F32)
    o1_ref[...] = t
    o2_ref[...] = jnp.concatenate([t[:, 0:D], t[:, 2 * D : 3 * D]], axis=1)


def _proj(h, w4, rows_blk):
    n = h.shape[0]
    grid = n // rows_blk
    return pl.pallas_call(
        _proj_body,
        grid=(grid,),
        in_specs=[
            pl.BlockSpec((rows_blk, D), lambda i: (i, 0)),
            pl.BlockSpec((D, 4 * D), lambda i: (0, 0)),
        ],
        out_specs=[
            pl.BlockSpec((rows_blk, 4 * D), lambda i: (i, 0)),
            pl.BlockSpec((rows_blk, 2 * D), lambda i: (i, 0)),
        ],
        out_shape=[
            jax.ShapeDtypeStruct((n, 4 * D), F32),
            jax.ShapeDtypeStruct((n, 2 * D), F32),
        ],
    )(h, w4)


def _mm_body(x_ref, w_ref, o_ref):
    o_ref[...] = jnp.dot(x_ref[...], w_ref[...], preferred_element_type=F32)


def _mm(x, w, rows_blk):
    n, k = x.shape
    m = w.shape[1]
    grid = n // rows_blk
    return pl.pallas_call(
        _mm_body,
        grid=(grid,),
        in_specs=[
            pl.BlockSpec((rows_blk, k), lambda i: (i, 0)),
            pl.BlockSpec((k, m), lambda i: (0, 0)),
        ],
        out_specs=pl.BlockSpec((rows_blk, m), lambda i: (i, 0)),
        out_shape=jax.ShapeDtypeStruct((n, m), F32),
    )(x, w)


def _edge_mlp_body(
    gs_ref, gd_ref, te_ref, dst_ref, wu0_ref, wu1_ref, bu0_ref, bu1_ref,
    wm0_ref, wm1_ref, bm0_ref, bm1_ref, o_ref,
):
    gs = gs_ref[...]
    gd = gd_ref[...]
    te = te_ref[...]
    wu_refs = (wu0_ref, wu1_ref)
    bu_refs = (bu0_ref, bu1_ref)
    wm_refs = (wm0_ref, wm1_ref)
    bm_refs = (bm0_ref, bm1_ref)
    for hd in range(2):
        q = gd[:, hd * D : (hd + 1) * D]
        k = gs[:, 2 * hd * D : (2 * hd + 1) * D]
        v = gs[:, (2 * hd + 1) * D : (2 * hd + 2) * D]
        ke = te[:, 2 * hd * D : (2 * hd + 1) * D]
        ve = te[:, (2 * hd + 1) * D : (2 * hd + 2) * D]
        a0 = q * k * (1.0 / 16.0)
        a1 = q * ke * (1.0 / 16.0)
        mu = (jnp.sum(a0, 1, keepdims=True) + jnp.sum(a1, 1, keepdims=True)) * (
            1.0 / (2.0 * D)
        )
        d0 = a0 - mu
        d1 = a1 - mu
        var = (jnp.sum(d0 * d0, 1, keepdims=True) + jnp.sum(d1 * d1, 1, keepdims=True)) * (
            1.0 / (2.0 * D)
        )
        sd = jnp.sqrt(var + 1e-5)
        g0 = jax.nn.sigmoid(d0 / sd)
        g1 = jax.nn.sigmoid(d1 / sd)
        wu = wu_refs[hd][...]
        u = (
            jnp.dot(v, wu[0:D, :], preferred_element_type=F32)
            + jnp.dot(ve, wu[D : 2 * D, :], preferred_element_type=F32)
            + bu_refs[hd][...]
        )
        wm = wm_refs[hd][...]
        y = (
            jnp.dot(u[:, 0:D] * g0, wm[0:D, :], preferred_element_type=F32)
            + jnp.dot(u[:, D : 2 * D] * g1, wm[D : 2 * D, :], preferred_element_type=F32)
            + bm_refs[hd][...]
        )
        mu2 = jnp.sum(y, 1, keepdims=True) * (1.0 / D)
        dy = y - mu2
        var2 = jnp.sum(dy * dy, 1, keepdims=True) * (1.0 / D)
        yn = dy / jnp.sqrt(var2 + 1e-5)
        o_ref[:, hd * D : (hd + 1) * D] = _leaky(yn)
    dst = dst_ref[...]
    dl = (dst - (dst // WPC) * WPC).astype(F32)
    o_ref[:, 2 * D : 2 * D + D] = dl * jnp.ones((1, D), F32)


def _edge_mlp(gsrc, gdst, te, dst, wu0, wu1, bu0, bu1, wm0, wm1, bm0, bm1,
              rows_blk):
    n = gsrc.shape[0]
    grid = n // rows_blk
    wspec = lambda shape: pl.BlockSpec(shape, lambda i: (0, 0))
    return pl.pallas_call(
        _edge_mlp_body,
        grid=(grid,),
        in_specs=[
            pl.BlockSpec((rows_blk, 4 * D), lambda i: (i, 0)),
            pl.BlockSpec((rows_blk, 2 * D), lambda i: (i, 0)),
            pl.BlockSpec((rows_blk, 4 * D), lambda i: (i, 0)),
            pl.BlockSpec((rows_blk, 1), lambda i: (i, 0)),
            wspec((2 * D, 2 * D)),
            wspec((2 * D, 2 * D)),
            wspec((1, 2 * D)),
            wspec((1, 2 * D)),
            wspec((2 * D, D)),
            wspec((2 * D, D)),
            wspec((1, D)),
            wspec((1, D)),
        ],
        out_specs=pl.BlockSpec((rows_blk, MW), lambda i: (i, 0)),
        out_shape=jax.ShapeDtypeStruct((n, MW), F32),
    )(
        gsrc, gdst, te, dst.reshape(n, 1), wu0, wu1,
        bu0.reshape(1, 2 * D), bu1.reshape(1, 2 * D),
        wm0, wm1, bm0.reshape(1, D), bm1.reshape(1, D),
    )


def _win_count_body(dst_ref, cnt_ref, st_ref):
    @pl.when(pl.program_id(0) == 0)
    def _():
        cnt_ref[...] = jnp.zeros_like(cnt_ref)
        st_ref[...] = jnp.zeros_like(st_ref)

    win = dst_ref[...] // WPC
    seg = lax.broadcasted_iota(jnp.int32, (1, NWINP), 1)
    oh = (win == seg).astype(F32)
    cnt_ref[0:1, :] += jnp.sum(oh, 0, keepdims=True)

    @pl.when(pl.program_id(0) == pl.num_programs(0) - 1)
    def _():
        counts = cnt_ref[0:1, :]
        i = lax.broadcasted_iota(jnp.int32, (NWINP, NWINP + 16), 0)
        j = lax.broadcasted_iota(jnp.int32, (NWINP, NWINP + 16), 1)
        m = (i < j).astype(F32)
        base = jnp.dot(counts, m, preferred_element_type=F32,
                       precision=lax.Precision.HIGHEST)
        st_ref[0:1, :] = base.astype(jnp.int32)


def _win_count(dst, rows_blk):
    n = dst.shape[0]
    grid = n // rows_blk
    return pl.pallas_call(
        _win_count_body,
        grid=(grid,),
        in_specs=[pl.BlockSpec((rows_blk, 1), lambda i: (i, 0))],
        out_specs=[
            pl.BlockSpec((8, NWINP), lambda i: (0, 0)),
            pl.BlockSpec((8, NWINP + 16), lambda i: (0, 0)),
        ],
        out_shape=[
            jax.ShapeDtypeStruct((8, NWINP), F32),
            jax.ShapeDtypeStruct((8, NWINP + 16), jnp.int32),
        ],
    )(dst.reshape(n, 1))


def _slot_body(dstc_ref, dstr_ref, st_ref, slot_ref, carry_ref):
    @pl.when(pl.program_id(0) == 0)
    def _():
        carry_ref[...] = jnp.zeros_like(carry_ref)

    b = dstc_ref.shape[0]
    win = jnp.floor(dstc_ref[...] * (1.0 / WPC))
    win_r = jnp.floor(dstr_ref[...] * (1.0 / WPC))
    seg = lax.broadcasted_iota(jnp.int32, (1, NWINP), 1).astype(F32)
    oh = (win == seg).astype(F32)
    eq = (win == win_r)
    qi = lax.broadcasted_iota(jnp.int32, (b, b), 1)
    pi = lax.broadcasted_iota(jnp.int32, (b, b), 0)
    low = qi < pi
    rank = jnp.sum((eq & low).astype(F32), axis=1, keepdims=True)
    base = st_ref[0:1, 0:NWINP].astype(F32) + carry_ref[0:1, :]
    dn = (((1,), (1,)), ((), ()))
    off = lax.dot_general(oh, base, dn, preferred_element_type=F32,
                          precision=lax.Precision.HIGHEST)
    slot_ref[...] = (off + rank).astype(jnp.int32)
    carry_ref[0:1, :] += jnp.sum(oh, 0, keepdims=True)


def _slot(dst, starts, rows_blk):
    n = dst.shape[0]
    grid = n // rows_blk
    dstf = dst.astype(F32)
    slot, _ = pl.pallas_call(
        _slot_body,
        grid=(grid,),
        in_specs=[
            pl.BlockSpec((rows_blk, 1), lambda i: (i, 0)),
            pl.BlockSpec((1, rows_blk), lambda i: (0, i)),
            pl.BlockSpec((8, NWINP + 16), lambda i: (0, 0)),
        ],
        out_specs=[
            pl.BlockSpec((rows_blk, 1), lambda i: (i, 0)),
            pl.BlockSpec((8, NWINP), lambda i: (0, 0)),
        ],
        out_shape=[
            jax.ShapeDtypeStruct((n, 1), jnp.int32),
            jax.ShapeDtypeStruct((8, NWINP), F32),
        ],
    )(dstf.reshape(n, 1), dstf.reshape(1, n), starts)
    return slot


def _cat_stats_body(hv_ref, w_ref, b_ref, y_ref, st_ref):
    y = jnp.dot(hv_ref[...], w_ref[...], preferred_element_type=F32) + b_ref[...]
    y_ref[...] = y

    @pl.when(pl.program_id(0) == 0)
    def _():
        st_ref[...] = jnp.zeros_like(st_ref)

    st_ref[0:1, :] += jnp.sum(y, 0, keepdims=True)
    st_ref[1:2, :] += jnp.sum(y * y, 0, keepdims=True)


def _cat_stats(hv, wcat, bcat, rows_blk):
    n = hv.shape[0]
    grid = n // rows_blk
    return pl.pallas_call(
        _cat_stats_body,
        grid=(grid,),
        in_specs=[
            pl.BlockSpec((rows_blk, 2 * D), lambda i: (i, 0)),
            pl.BlockSpec((2 * D, D), lambda i: (0, 0)),
            pl.BlockSpec((1, D), lambda i: (0, 0)),
        ],
        out_specs=[
            pl.BlockSpec((rows_blk, D), lambda i: (i, 0)),
            pl.BlockSpec((8, D), lambda i: (0, 0)),
        ],
        out_shape=[
            jax.ShapeDtypeStruct((n, D), F32),
            jax.ShapeDtypeStruct((8, D), F32),
        ],
    )(hv, wcat, bcat.reshape(1, D))


def _bn_res_body(y_ref, st_ref, h_ref, o_ref, *, n):
    st = st_ref[...]
    m = st[0:1, :] * (1.0 / n)
    var = st[1:2, :] * (1.0 / n) - m * m
    o_ref[...] = _leaky((y_ref[...] - m) / jnp.sqrt(var + 1e-5)) + h_ref[...]


def _bn_res(y, stats, h, rows_blk):
    n = y.shape[0]
    grid = n // rows_blk
    body = functools.partial(_bn_res_body, n=float(n))
    return pl.pallas_call(
        body,
        grid=(grid,),
        in_specs=[
            pl.BlockSpec((rows_blk, D), lambda i: (i, 0)),
            pl.BlockSpec((8, D), lambda i: (0, 0)),
            pl.BlockSpec((rows_blk, D), lambda i: (i, 0)),
        ],
        out_specs=pl.BlockSpec((rows_blk, D), lambda i: (i, 0)),
        out_shape=jax.ShapeDtypeStruct((n, D), F32),
    )(y, stats, h)


def _pool_body(h_ref, b_ref, s_ref, c_ref):
    @pl.when(pl.program_id(0) == 0)
    def _():
        s_ref[...] = jnp.zeros_like(s_ref)
        c_ref[...] = jnp.zeros_like(c_ref)

    h = h_ref[...]
    bids = b_ref[...]
    seg = lax.broadcasted_iota(jnp.int32, (1, 64), 1)
    oh = (bids == seg).astype(F32)
    dn = (((0,), (0,)), ((), ()))
    s_ref[...] += lax.dot_general(oh, h, dn, preferred_element_type=F32,
                                  precision=lax.Precision.HIGHEST)
    c_ref[...] += lax.dot_general(oh, jnp.ones_like(h), dn, preferred_element_type=F32,
                                  precision=lax.Precision.HIGHEST)


def _pool(h, bids, rows_blk):
    n = h.shape[0]
    grid = n // rows_blk
    return pl.pallas_call(
        _pool_body,
        grid=(grid,),
        in_specs=[
            pl.BlockSpec((rows_blk, D), lambda i: (i, 0)),
            pl.BlockSpec((rows_blk, 1), lambda i: (i, 0)),
        ],
        out_specs=[
            pl.BlockSpec((64, D), lambda i: (0, 0)),
            pl.BlockSpec((64, D), lambda i: (0, 0)),
        ],
        out_shape=[
            jax.ShapeDtypeStruct((64, D), F32),
            jax.ShapeDtypeStruct((64, D), F32),
        ],
    )(h, bids.reshape(n, 1))


def _head_body(s1_ref, c1_ref, s2_ref, c2_ref, wfc_ref, bfc_ref, wfc2_ref,
               bfc2_ref, wo_ref, bo_ref, o_ref):
    f1 = s1_ref[...] / jnp.maximum(c1_ref[...], 1.0)
    f2 = s2_ref[...] / jnp.maximum(c2_ref[...], 1.0)
    wfc = wfc_ref[...]
    a = (
        jnp.dot(f1, wfc[0:D, :], preferred_element_type=F32)
        + jnp.dot(f2, wfc[D : 2 * D, :], preferred_element_type=F32)
        + bfc_ref[...]
    )
    a = _leaky(a)
    a = _leaky(jnp.dot(a, wfc2_ref[...], preferred_element_type=F32) + bfc2_ref[...])
    o_ref[...] = jnp.dot(a, wo_ref[...], preferred_element_type=F32) + bo_ref[...]


def _head(s1, c1, s2, c2, wfc, bfc, wfc2, bfc2, wo, bo):
    wo_pad = jnp.zeros((D, D), F32).at[:, 0].set(wo[:, 0])
    bo_pad = jnp.zeros((1, D), F32).at[0, 0].set(bo[0])
    full = lambda a: pl.BlockSpec(a.shape, lambda: (0,) * a.ndim)
    args = (s1, c1, s2, c2, wfc, bfc.reshape(1, D), wfc2, bfc2.reshape(1, D),
            wo_pad, bo_pad)
    out = pl.pallas_call(
        _head_body,
        in_specs=[full(a) for a in args],
        out_specs=pl.BlockSpec((64, D), lambda: (0, 0)),
        out_shape=jax.ShapeDtypeStruct((64, D), F32),
    )(*args)
    return out[:, 0]


_GATHER_CHUNK = 128
_ACC_CHUNK = 64


def _sc_gather(table, idx):
    n, width = table.shape
    e = idx.shape[0]
    nch = e // _GATHER_CHUNK
    mesh = plsc.VectorSubcoreMesh(
        core_axis_name="c", subcore_axis_name="s", num_cores=NC, num_subcores=NS
    )

    @functools.partial(
        pl.kernel,
        out_type=jax.ShapeDtypeStruct((e, width), F32),
        mesh=mesh,
        scratch_types=[
            pltpu.VMEM((_GATHER_CHUNK,), jnp.int32),
            pltpu.VMEM((_GATHER_CHUNK, width), F32),
            pltpu.SemaphoreType.DMA,
        ],
    )
    def k(tab_hbm, idx_hbm, out_hbm, idx_v, rows_v, sem):
        wid = lax.axis_index("s") * NC + lax.axis_index("c")
        trips = (nch - wid + NW - 1) // NW

        def body(i, carry):
            base = (wid + i * NW) * _GATHER_CHUNK
            pltpu.sync_copy(idx_hbm.at[pl.ds(base, _GATHER_CHUNK)], idx_v)
            pltpu.async_copy(tab_hbm.at[idx_v], rows_v, sem).wait()
            pltpu.sync_copy(rows_v, out_hbm.at[pl.ds(base, _GATHER_CHUNK)])
            return carry

        lax.fori_loop(0, trips, body, 0)

    return k(table, idx)


def _sc_regroup(msgs, slot):
    e = msgs.shape[0]
    nch = e // _GATHER_CHUNK
    mesh = plsc.VectorSubcoreMesh(
        core_axis_name="c", subcore_axis_name="s", num_cores=NC, num_subcores=NS
    )

    @functools.partial(
        pl.kernel,
        out_type=jax.ShapeDtypeStruct((e + _ACC_CHUNK, MW), F32),
        mesh=mesh,
        scratch_types=[
            pltpu.VMEM((_GATHER_CHUNK,), jnp.int32),
            pltpu.VMEM((_GATHER_CHUNK, MW), F32),
        ],
    )
    def k(msgs_hbm, slot_hbm, out_hbm, idx_v, rows_v):
        wid = lax.axis_index("s") * NC + lax.axis_index("c")
        trips = (nch - wid + NW - 1) // NW

        def body(i, carry):
            base = (wid + i * NW) * _GATHER_CHUNK
            pltpu.sync_copy(slot_hbm.at[pl.ds(base, _GATHER_CHUNK)], idx_v)
            pltpu.sync_copy(msgs_hbm.at[pl.ds(base, _GATHER_CHUNK)], rows_v)
            pltpu.sync_copy(rows_v, out_hbm.at[idx_v])
            return carry

        lax.fori_loop(0, trips, body, 0)

    return k(msgs, slot)


def _sc_win_accum(grouped_flat, starts, w_iter):
    mesh = plsc.VectorSubcoreMesh(
        core_axis_name="c", subcore_axis_name="s", num_cores=NC, num_subcores=NS
    )
    wrows = WPC * 256

    @functools.partial(
        pl.kernel,
        out_type=jax.ShapeDtypeStruct((w_iter * wrows,), F32),
        mesh=mesh,
        scratch_types=[
            pltpu.VMEM((NWINP + 16,), jnp.int32),
            pltpu.VMEM((_ACC_CHUNK * MW,), F32),
            pltpu.VMEM((wrows,), F32),
            pltpu.SemaphoreType.DMA,
        ],
    )
    def k(g_hbm, st_hbm, out_hbm, starts_v, chunk_v, win_v, sem):
        wid = lax.axis_index("s") * NC + lax.axis_index("c")
        pltpu.sync_copy(st_hbm, starts_v)
        trips = (w_iter - wid + NW - 1) // NW

        def wloop(t, c0):
            w = wid + t * NW
            sv = starts_v[pl.ds(w, 16)]
            s0 = sv[0]
            cnt = sv[1] - s0

            def zloop(z, c1):
                win_v[pl.ds(z * 16, 16)] = jnp.zeros((16,), F32)
                return c1

            lax.fori_loop(0, wrows // 16, zloop, 0)

            def chunk_loop(ch, c2):
                pltpu.sync_copy(
                    g_hbm.at[pl.ds((s0 + ch * _ACC_CHUNK) * MW, _ACC_CHUNK * MW)],
                    chunk_v,
                )
                nrows = jnp.minimum(jnp.int32(_ACC_CHUNK), cnt - ch * _ACC_CHUNK)

                def row_loop(j, c3):
                    dl = chunk_v[pl.ds(j * MW + 256, 16)][0].astype(jnp.int32)

                    def kloop(kk, c4):
                        off = dl * 256 + kk * 16
                        win_v[pl.ds(off, 16)] = (
                            win_v[pl.ds(off, 16)]
                            + chunk_v[pl.ds(j * MW + kk * 16, 16)]
                        )
                        return c4

                    lax.fori_loop(0, 16, kloop, 0)
                    return c3

                lax.fori_loop(0, nrows, row_loop, 0)
                return c2

            lax.fori_loop(0, (cnt + _ACC_CHUNK - 1) // _ACC_CHUNK, chunk_loop, 0)

            def oloop(sub, c5):
                pltpu.sync_copy(
                    win_v.at[pl.ds(sub * 4096, 4096)],
                    out_hbm.at[pl.ds(w * wrows + sub * 4096, 4096)],
                )
                return c5

            lax.fori_loop(0, wrows // 4096, oloop, 0)
            return c0

        lax.fori_loop(0, trips, wloop, 0)

    return k(grouped_flat, starts)


def _sc_scatter_add(msgs, dst, n_out):
    nwin = -(-n_out // WPC)
    w_iter = -(-nwin // NW) * NW
    _, starts = _win_count(dst, 2000)
    slot = _slot(dst, starts, 640)
    grouped = _sc_regroup(msgs, slot.reshape(-1))
    out_flat = _sc_win_accum(
        grouped.reshape(-1), starts[0].astype(jnp.int32), w_iter
    )
    return out_flat.reshape(w_iter * WPC, 256)[:n_out]


def _qcconv(c, h, src, dst, efeat, n_nodes, P, rows_blk):
    wn = jnp.concatenate(
        [P["Wkv"][c][0], P["Wvv"][c][0], P["Wkv"][c][1], P["Wvv"][c][1]], axis=1
    )
    we4 = jnp.concatenate(
        [P["Wke"][c][0], P["Wve"][c][0], P["Wke"][c][1], P["Wve"][c][1]], axis=1
    )
    t1, t2 = _proj(h, wn, rows_blk)
    te = _mm(efeat, we4, 2000)
    gsrc = _sc_gather(t1, src)
    gdst = _sc_gather(t2, dst)
    msgs = _edge_mlp(
        gsrc, gdst, te, dst,
        P["Wu"][c][0], P["Wu"][c][1], P["bu"][c][0], P["bu"][c][1],
        P["Wm"][c][0], P["Wm"][c][1], P["bm"][c][0], P["bm"][c][1],
        1000,
    )
    hv = _sc_scatter_add(msgs, dst, n_nodes)
    y, stats = _cat_stats(hv, P["Wcat"][c], P["bcat"][c], rows_blk)
    return _bn_res(y, stats, h, rows_blk)


def kernel(x, edge_index, edge_dis, triangle_index, triangle_dis, x_batch,
           edge_dis_batch, Wkv, Wke, Wvv, Wve, Wu, bu, Wm, bm, Wcat, bcat, Wa,
           ba, We, be, Wt, bt, Wfc, bfc, Wfc2, bfc2, Wo, bo):
    P = {"Wkv": Wkv, "Wke": Wke, "Wvv": Wvv, "Wve": Wve, "Wu": Wu, "bu": bu,
         "Wm": Wm, "bm": bm, "Wcat": Wcat, "bcat": bcat}
    src_t = triangle_index[0]
    dst_t = triangle_index[1]
    src_e = edge_index[0]
    dst_e = edge_index[1]

    h0 = _node_embed(x, Wa, ba, 1000)
    ef0 = _feat_embed(edge_dis, We, be, 64, 2000)
    tf = _feat_embed(triangle_dis, Wt, bt, 80, 2000)

    ef1 = _qcconv(0, ef0, src_t, dst_t, tf, E, P, 2000)
    h1 = _qcconv(1, h0, src_e, dst_e, ef1, N, P, 1000)

    s1, c1 = _pool(h1, x_batch, 1000)
    s2, c2 = _pool(ef1, edge_dis_batch, 2000)
    return _head(s1, c1, s2, c2, Wfc, bfc, Wfc2, bfc2, Wo, bo)


# speedup vs baseline: 1.0580x; 1.0580x over previous
import functools

import jax
import jax.numpy as jnp
from jax import lax
from jax.experimental import pallas as pl
from jax.experimental.pallas import tpu as pltpu
from jax.experimental.pallas import tpu_sc as plsc

N = 10000
E = 160000
D = 128

NC = 2
NS = 16
NW = NC * NS

WPC = 256
NWINP = 768
MW = 384

F32 = jnp.float32


def _leaky(v):
    return jnp.where(v >= 0, v, 0.01 * v)


def _node_embed_body(x_ref, w_ref, b_ref, o_ref):
    o_ref[...] = (
        jnp.dot(x_ref[...], w_ref[...], preferred_element_type=F32) + b_ref[...]
    )


def _node_embed(x, w, b, rows_blk):
    n = x.shape[0]
    grid = n // rows_blk
    return pl.pallas_call(
        _node_embed_body,
        grid=(grid,),
        in_specs=[
            pl.BlockSpec((rows_blk, x.shape[1]), lambda i: (i, 0)),
            pl.BlockSpec(w.shape, lambda i: (0, 0)),
            pl.BlockSpec((1, D), lambda i: (0, 0)),
        ],
        out_specs=pl.BlockSpec((rows_blk, D), lambda i: (i, 0)),
        out_shape=jax.ShapeDtypeStruct((n, D), F32),
    )(x, w, b.reshape(1, D))


def _feat_embed_body(d_ref, c_ref, w_ref, b_ref, o_ref, *, bins, gamma):
    d = d_ref[...]
    c = c_ref[...]
    w = w_ref[...]
    acc = jnp.zeros((d.shape[0], D), F32)
    for i in range(3):
        di = d[:, i : i + 1]
        r = jnp.exp(-gamma * (di - c) ** 2)
        acc += jnp.dot(r, w[i * bins : (i + 1) * bins, :], preferred_element_type=F32)
    acc += jnp.dot(d[:, 3:], w[3 * bins :, :], preferred_element_type=F32)
    o_ref[...] = acc + b_ref[...]


def _feat_embed(d, w, b, bins, rows_blk):
    n, f = d.shape
    gamma = 1.0 / (8.0 / bins) ** 2
    centers = jnp.linspace(0.0, 8.0, bins, dtype=F32).reshape(1, bins)
    grid = n // rows_blk
    body = functools.partial(_feat_embed_body, bins=bins, gamma=gamma)
    return pl.pallas_call(
        body,
        grid=(grid,),
        in_specs=[
            pl.BlockSpec((rows_blk, f), lambda i: (i, 0)),
            pl.BlockSpec((1, bins), lambda i: (0, 0)),
            pl.BlockSpec(w.shape, lambda i: (0, 0)),
            pl.BlockSpec((1, D), lambda i: (0, 0)),
        ],
        out_specs=pl.BlockSpec((rows_blk, D), lambda i: (i, 0)),
        out_shape=jax.ShapeDtypeStruct((n, D), F32),
    )(d, centers, w, b.reshape(1, D))


def _proj_body(h_ref, w_ref, o1_ref, o2_ref):
    t = jnp.dot(h_ref[...], w_ref[...], preferred_element_type=F32)
    o1_ref[...] = t
    o2_ref[...] = jnp.concatenate([t[:, 0:D], t[:, 2 * D : 3 * D]], axis=1)


def _proj(h, w4, rows_blk):
    n = h.shape[0]
    grid = n // rows_blk
    return pl.pallas_call(
        _proj_body,
        grid=(grid,),
        in_specs=[
            pl.BlockSpec((rows_blk, D), lambda i: (i, 0)),
            pl.BlockSpec((D, 4 * D), lambda i: (0, 0)),
        ],
        out_specs=[
            pl.BlockSpec((rows_blk, 4 * D), lambda i: (i, 0)),
            pl.BlockSpec((rows_blk, 2 * D), lambda i: (i, 0)),
        ],
        out_shape=[
            jax.ShapeDtypeStruct((n, 4 * D), F32),
            jax.ShapeDtypeStruct((n, 2 * D), F32),
        ],
    )(h, w4)


def _mm_body(x_ref, w_ref, o_ref):
    o_ref[...] = jnp.dot(x_ref[...], w_ref[...], preferred_element_type=F32)


def _mm(x, w, rows_blk):
    n, k = x.shape
    m = w.shape[1]
    grid = n // rows_blk
    return pl.pallas_call(
        _mm_body,
        grid=(grid,),
        in_specs=[
            pl.BlockSpec((rows_blk, k), lambda i: (i, 0)),
            pl.BlockSpec((k, m), lambda i: (0, 0)),
        ],
        out_specs=pl.BlockSpec((rows_blk, m), lambda i: (i, 0)),
        out_shape=jax.ShapeDtypeStruct((n, m), F32),
    )(x, w)


def _edge_mlp_body(
    gs_ref, gd_ref, te_ref, dst_ref, wu0_ref, wu1_ref, bu0_ref, bu1_ref,
    wm0_ref, wm1_ref, bm0_ref, bm1_ref, o_ref,
):
    gs = gs_ref[...]
    gd = gd_ref[...]
    te = te_ref[...]
    wu_refs = (wu0_ref, wu1_ref)
    bu_refs = (bu0_ref, bu1_ref)
    wm_refs = (wm0_ref, wm1_ref)
    bm_refs = (bm0_ref, bm1_ref)
    for hd in range(2):
        q = gd[:, hd * D : (hd + 1) * D]
        k = gs[:, 2 * hd * D : (2 * hd + 1) * D]
        v = gs[:, (2 * hd + 1) * D : (2 * hd + 2) * D]
        ke = te[:, 2 * hd * D : (2 * hd + 1) * D]
        ve = te[:, (2 * hd + 1) * D : (2 * hd + 2) * D]
        a0 = q * k * (1.0 / 16.0)
        a1 = q * ke * (1.0 / 16.0)
        mu = (jnp.sum(a0, 1, keepdims=True) + jnp.sum(a1, 1, keepdims=True)) * (
            1.0 / (2.0 * D)
        )
        d0 = a0 - mu
        d1 = a1 - mu
        var = (jnp.sum(d0 * d0, 1, keepdims=True) + jnp.sum(d1 * d1, 1, keepdims=True)) * (
            1.0 / (2.0 * D)
        )
        sd = jnp.sqrt(var + 1e-5)
        g0 = jax.nn.sigmoid(d0 / sd)
        g1 = jax.nn.sigmoid(d1 / sd)
        wu = wu_refs[hd][...]
        u = (
            jnp.dot(v, wu[0:D, :], preferred_element_type=F32)
            + jnp.dot(ve, wu[D : 2 * D, :], preferred_element_type=F32)
            + bu_refs[hd][...]
        )
        wm = wm_refs[hd][...]
        y = (
            jnp.dot(u[:, 0:D] * g0, wm[0:D, :], preferred_element_type=F32)
            + jnp.dot(u[:, D : 2 * D] * g1, wm[D : 2 * D, :], preferred_element_type=F32)
            + bm_refs[hd][...]
        )
        mu2 = jnp.sum(y, 1, keepdims=True) * (1.0 / D)
        dy = y - mu2
        var2 = jnp.sum(dy * dy, 1, keepdims=True) * (1.0 / D)
        yn = dy / jnp.sqrt(var2 + 1e-5)
        o_ref[:, hd * D : (hd + 1) * D] = _leaky(yn)
    dst = dst_ref[...]
    dl = (dst - (dst // WPC) * WPC).astype(F32)
    o_ref[:, 2 * D : 2 * D + D] = dl * jnp.ones((1, D), F32)


def _edge_mlp(gsrc, gdst, te, dst, wu0, wu1, bu0, bu1, wm0, wm1, bm0, bm1,
              rows_blk):
    n = gsrc.shape[0]
    grid = n // rows_blk
    wspec = lambda shape: pl.BlockSpec(shape, lambda i: (0, 0))
    return pl.pallas_call(
        _edge_mlp_body,
        grid=(grid,),
        in_specs=[
            pl.BlockSpec((rows_blk, 4 * D), lambda i: (i, 0)),
            pl.BlockSpec((rows_blk, 2 * D), lambda i: (i, 0)),
            pl.BlockSpec((rows_blk, 4 * D), lambda i: (i, 0)),
            pl.BlockSpec((rows_blk, 1), lambda i: (i, 0)),
            wspec((2 * D, 2 * D)),
            wspec((2 * D, 2 * D)),
            wspec((1, 2 * D)),
            wspec((1, 2 * D)),
            wspec((2 * D, D)),
            wspec((2 * D, D)),
            wspec((1, D)),
            wspec((1, D)),
        ],
        out_specs=pl.BlockSpec((rows_blk, MW), lambda i: (i, 0)),
        out_shape=jax.ShapeDtypeStruct((n, MW), F32),
    )(
        gsrc, gdst, te, dst.reshape(n, 1), wu0, wu1,
        bu0.reshape(1, 2 * D), bu1.reshape(1, 2 * D),
        wm0, wm1, bm0.reshape(1, D), bm1.reshape(1, D),
    )


def _win_count_body(dst_ref, cnt_ref, st_ref):
    @pl.when(pl.program_id(0) == 0)
    def _():
        cnt_ref[...] = jnp.zeros_like(cnt_ref)
        st_ref[...] = jnp.zeros_like(st_ref)

    win = dst_ref[...] // WPC
    seg = lax.broadcasted_iota(jnp.int32, (1, NWINP), 1)
    oh = (win == seg).astype(F32)
    cnt_ref[0:1, :] += jnp.sum(oh, 0, keepdims=True)

    @pl.when(pl.program_id(0) == pl.num_programs(0) - 1)
    def _():
        counts = cnt_ref[0:1, :]
        i = lax.broadcasted_iota(jnp.int32, (NWINP, NWINP + 16), 0)
        j = lax.broadcasted_iota(jnp.int32, (NWINP, NWINP + 16), 1)
        m = (i < j).astype(F32)
        base = jnp.dot(counts, m, preferred_element_type=F32,
                       precision=lax.Precision.HIGHEST)
        st_ref[0:1, :] = base.astype(jnp.int32)


def _win_count(dst, rows_blk):
    n = dst.shape[0]
    grid = n // rows_blk
    return pl.pallas_call(
        _win_count_body,
        grid=(grid,),
        in_specs=[pl.BlockSpec((rows_blk, 1), lambda i: (i, 0))],
        out_specs=[
            pl.BlockSpec((8, NWINP), lambda i: (0, 0)),
            pl.BlockSpec((8, NWINP + 16), lambda i: (0, 0)),
        ],
        out_shape=[
            jax.ShapeDtypeStruct((8, NWINP), F32),
            jax.ShapeDtypeStruct((8, NWINP + 16), jnp.int32),
        ],
    )(dst.reshape(n, 1))


def _slot_body(dstc_ref, dstr_ref, st_ref, slot_ref, carry_ref):
    @pl.when(pl.program_id(0) == 0)
    def _():
        carry_ref[...] = jnp.zeros_like(carry_ref)

    b = dstc_ref.shape[0]
    win = jnp.floor(dstc_ref[...] * (1.0 / WPC))
    win_r = jnp.floor(dstr_ref[...] * (1.0 / WPC))
    seg = lax.broadcasted_iota(jnp.int32, (1, NWINP), 1).astype(F32)
    oh = (win == seg).astype(F32)
    eq = (win == win_r)
    qi = lax.broadcasted_iota(jnp.int32, (b, b), 1)
    pi = lax.broadcasted_iota(jnp.int32, (b, b), 0)
    low = qi < pi
    rank = jnp.sum((eq & low).astype(F32), axis=1, keepdims=True)
    base = st_ref[0:1, 0:NWINP].astype(F32) + carry_ref[0:1, :]
    dn = (((1,), (1,)), ((), ()))
    off = lax.dot_general(oh, base, dn, preferred_element_type=F32,
                          precision=lax.Precision.HIGHEST)
    slot_ref[...] = (off + rank).astype(jnp.int32)
    carry_ref[0:1, :] += jnp.sum(oh, 0, keepdims=True)


def _slot(dst, starts, rows_blk):
    n = dst.shape[0]
    grid = n // rows_blk
    dstf = dst.astype(F32)
    slot, _ = pl.pallas_call(
        _slot_body,
        grid=(grid,),
        in_specs=[
            pl.BlockSpec((rows_blk, 1), lambda i: (i, 0)),
            pl.BlockSpec((1, rows_blk), lambda i: (0, i)),
            pl.BlockSpec((8, NWINP + 16), lambda i: (0, 0)),
        ],
        out_specs=[
            pl.BlockSpec((rows_blk, 1), lambda i: (i, 0)),
            pl.BlockSpec((8, NWINP), lambda i: (0, 0)),
        ],
        out_shape=[
            jax.ShapeDtypeStruct((n, 1), jnp.int32),
            jax.ShapeDtypeStruct((8, NWINP), F32),
        ],
    )(dstf.reshape(n, 1), dstf.reshape(1, n), starts)
    return slot


def _cat_stats_body(hv_ref, w_ref, b_ref, y_ref, st_ref):
    y = jnp.dot(hv_ref[...], w_ref[...], preferred_element_type=F32) + b_ref[...]
    y_ref[...] = y

    @pl.when(pl.program_id(0) == 0)
    def _():
        st_ref[...] = jnp.zeros_like(st_ref)

    st_ref[0:1, :] += jnp.sum(y, 0, keepdims=True)
    st_ref[1:2, :] += jnp.sum(y * y, 0, keepdims=True)


def _cat_stats(hv, wcat, bcat, rows_blk):
    n = hv.shape[0]
    grid = n // rows_blk
    return pl.pallas_call(
        _cat_stats_body,
        grid=(grid,),
        in_specs=[
            pl.BlockSpec((rows_blk, 2 * D), lambda i: (i, 0)),
            pl.BlockSpec((2 * D, D), lambda i: (0, 0)),
            pl.BlockSpec((1, D), lambda i: (0, 0)),
        ],
        out_specs=[
            pl.BlockSpec((rows_blk, D), lambda i: (i, 0)),
            pl.BlockSpec((8, D), lambda i: (0, 0)),
        ],
        out_shape=[
            jax.ShapeDtypeStruct((n, D), F32),
            jax.ShapeDtypeStruct((8, D), F32),
        ],
    )(hv, wcat, bcat.reshape(1, D))


def _bn_res_body(y_ref, st_ref, h_ref, o_ref, *, n):
    st = st_ref[...]
    m = st[0:1, :] * (1.0 / n)
    var = st[1:2, :] * (1.0 / n) - m * m
    o_ref[...] = _leaky((y_ref[...] - m) / jnp.sqrt(var + 1e-5)) + h_ref[...]


def _bn_res(y, stats, h, rows_blk):
    n = y.shape[0]
    grid = n // rows_blk
    body = functools.partial(_bn_res_body, n=float(n))
    return pl.pallas_call(
        body,
        grid=(grid,),
        in_specs=[
            pl.BlockSpec((rows_blk, D), lambda i: (i, 0)),
            pl.BlockSpec((8, D), lambda i: (0, 0)),
            pl.BlockSpec((rows_blk, D), lambda i: (i, 0)),
        ],
        out_specs=pl.BlockSpec((rows_blk, D), lambda i: (i, 0)),
        out_shape=jax.ShapeDtypeStruct((n, D), F32),
    )(y, stats, h)


def _pool_body(h_ref, b_ref, s_ref, c_ref):
    @pl.when(pl.program_id(0) == 0)
    def _():
        s_ref[...] = jnp.zeros_like(s_ref)
        c_ref[...] = jnp.zeros_like(c_ref)

    h = h_ref[...]
    bids = b_ref[...]
    seg = lax.broadcasted_iota(jnp.int32, (1, 64), 1)
    oh = (bids == seg).astype(F32)
    dn = (((0,), (0,)), ((), ()))
    s_ref[...] += lax.dot_general(oh, h, dn, preferred_element_type=F32,
                                  precision=lax.Precision.HIGHEST)
    c_ref[...] += lax.dot_general(oh, jnp.ones_like(h), dn, preferred_element_type=F32,
                                  precision=lax.Precision.HIGHEST)


def _pool(h, bids, rows_blk):
    n = h.shape[0]
    grid = n // rows_blk
    return pl.pallas_call(
        _pool_body,
        grid=(grid,),
        in_specs=[
            pl.BlockSpec((rows_blk, D), lambda i: (i, 0)),
            pl.BlockSpec((rows_blk, 1), lambda i: (i, 0)),
        ],
        out_specs=[
            pl.BlockSpec((64, D), lambda i: (0, 0)),
            pl.BlockSpec((64, D), lambda i: (0, 0)),
        ],
        out_shape=[
            jax.ShapeDtypeStruct((64, D), F32),
            jax.ShapeDtypeStruct((64, D), F32),
        ],
    )(h, bids.reshape(n, 1))


def _head_body(s1_ref, c1_ref, s2_ref, c2_ref, wfc_ref, bfc_ref, wfc2_ref,
               bfc2_ref, wo_ref, bo_ref, o_ref):
    f1 = s1_ref[...] / jnp.maximum(c1_ref[...], 1.0)
    f2 = s2_ref[...] / jnp.maximum(c2_ref[...], 1.0)
    wfc = wfc_ref[...]
    a = (
        jnp.dot(f1, wfc[0:D, :], preferred_element_type=F32)
        + jnp.dot(f2, wfc[D : 2 * D, :], preferred_element_type=F32)
        + bfc_ref[...]
    )
    a = _leaky(a)
    a = _leaky(jnp.dot(a, wfc2_ref[...], preferred_element_type=F32) + bfc2_ref[...])
    o_ref[...] = jnp.dot(a, wo_ref[...], preferred_element_type=F32) + bo_ref[...]


def _head(s1, c1, s2, c2, wfc, bfc, wfc2, bfc2, wo, bo):
    wo_pad = jnp.zeros((D, D), F32).at[:, 0].set(wo[:, 0])
    bo_pad = jnp.zeros((1, D), F32).at[0, 0].set(bo[0])
    full = lambda a: pl.BlockSpec(a.shape, lambda: (0,) * a.ndim)
    args = (s1, c1, s2, c2, wfc, bfc.reshape(1, D), wfc2, bfc2.reshape(1, D),
            wo_pad, bo_pad)
    out = pl.pallas_call(
        _head_body,
        in_specs=[full(a) for a in args],
        out_specs=pl.BlockSpec((64, D), lambda: (0, 0)),
        out_shape=jax.ShapeDtypeStruct((64, D), F32),
    )(*args)
    return out[:, 0]


_GATHER_CHUNK = 128
_ACC_CHUNK = 64


_GC = 40


def _sc_gather(table, idx):
    n, width = table.shape
    e = idx.shape[0]
    per_w = e // NW
    nch = per_w // _GC
    mesh = plsc.VectorSubcoreMesh(
        core_axis_name="c", subcore_axis_name="s", num_cores=NC, num_subcores=NS
    )

    @functools.partial(
        pl.kernel,
        out_type=jax.ShapeDtypeStruct((e, width), F32),
        mesh=mesh,
        scratch_types=[
            pltpu.VMEM((per_w,), jnp.int32),
            pltpu.VMEM((_GC, width), F32),
            pltpu.VMEM((_GC, width), F32),
            pltpu.SemaphoreType.DMA,
            pltpu.SemaphoreType.DMA,
            pltpu.SemaphoreType.DMA,
            pltpu.SemaphoreType.DMA,
        ],
    )
    def k(tab_hbm, idx_hbm, out_hbm, idx_v, r0, r1, sg0, sg1, sw0, sw1):
        wid = lax.axis_index("s") * NC + lax.axis_index("c")
        w0 = wid * per_w
        pltpu.sync_copy(idx_hbm.at[pl.ds(w0, per_w)], idx_v)
        bufs = (r0, r1)
        sgs = (sg0, sg1)
        sws = (sw0, sw1)

        def start_gather(c, b):
            pltpu.async_copy(
                tab_hbm.at[idx_v.at[pl.ds(c * _GC, _GC)]], bufs[b], sgs[b]
            )

        def wait_gather(b):
            pltpu.make_async_copy(tab_hbm.at[idx_v.at[pl.ds(0, _GC)]], bufs[b],
                                  sgs[b]).wait()

        def start_write(c, b):
            pltpu.async_copy(bufs[b], out_hbm.at[pl.ds(w0 + c * _GC, _GC)], sws[b])

        def wait_write(b):
            pltpu.make_async_copy(bufs[b], out_hbm.at[pl.ds(w0, _GC)], sws[b]).wait()

        start_gather(0, 0)

        def pair(i, carry):
            for b in range(2):
                c = i * 2 + b
                nb = 1 - b

                @pl.when(c + 1 < nch)
                def _():
                    @pl.when(c >= 1)
                    def _():
                        wait_write(nb)

                    start_gather(c + 1, nb)

                wait_gather(b)
                start_write(c, b)
            return carry

        lax.fori_loop(0, nch // 2, pair, 0)
        if nch % 2:
            c = nch - 1
            wait_gather(c % 2)
            start_write(c, c % 2)
        wait_write(0)
        wait_write(1)

    return k(table, idx)


def _sc_regroup(msgs, slot):
    e = msgs.shape[0]
    per_w = e // NW
    nch = per_w // _GC
    mesh = plsc.VectorSubcoreMesh(
        core_axis_name="c", subcore_axis_name="s", num_cores=NC, num_subcores=NS
    )

    @functools.partial(
        pl.kernel,
        out_type=jax.ShapeDtypeStruct((e + _ACC_CHUNK, MW), F32),
        mesh=mesh,
        scratch_types=[
            pltpu.VMEM((per_w,), jnp.int32),
            pltpu.VMEM((_GC, MW), F32),
            pltpu.VMEM((_GC, MW), F32),
            pltpu.SemaphoreType.DMA,
            pltpu.SemaphoreType.DMA,
            pltpu.SemaphoreType.DMA,
            pltpu.SemaphoreType.DMA,
        ],
    )
    def k(msgs_hbm, slot_hbm, out_hbm, idx_v, r0, r1, sg0, sg1, sw0, sw1):
        wid = lax.axis_index("s") * NC + lax.axis_index("c")
        w0 = wid * per_w
        pltpu.sync_copy(slot_hbm.at[pl.ds(w0, per_w)], idx_v)
        bufs = (r0, r1)
        sgs = (sg0, sg1)
        sws = (sw0, sw1)

        def start_read(c, b):
            pltpu.async_copy(msgs_hbm.at[pl.ds(w0 + c * _GC, _GC)], bufs[b], sgs[b])

        def wait_read(b):
            pltpu.make_async_copy(msgs_hbm.at[pl.ds(w0, _GC)], bufs[b], sgs[b]).wait()

        def start_write(c, b):
            pltpu.async_copy(
                bufs[b], out_hbm.at[idx_v.at[pl.ds(c * _GC, _GC)]], sws[b]
            )

        def wait_write(b):
            pltpu.make_async_copy(
                bufs[b], out_hbm.at[idx_v.at[pl.ds(0, _GC)]], sws[b]
            ).wait()

        start_read(0, 0)

        def pair(i, carry):
            for b in range(2):
                c = i * 2 + b
                nb = 1 - b

                @pl.when(c + 1 < nch)
                def _():
                    @pl.when(c >= 1)
                    def _():
                        wait_write(nb)

                    start_read(c + 1, nb)

                wait_read(b)
                start_write(c, b)
            return carry

        lax.fori_loop(0, nch // 2, pair, 0)
        if nch % 2:
            c = nch - 1
            wait_read(c % 2)
            start_write(c, c % 2)
        wait_write(0)
        wait_write(1)

    return k(msgs, slot)


def _sc_win_accum(grouped_flat, starts, w_iter):
    mesh = plsc.VectorSubcoreMesh(
        core_axis_name="c", subcore_axis_name="s", num_cores=NC, num_subcores=NS
    )
    wrows = WPC * 256

    @functools.partial(
        pl.kernel,
        out_type=jax.ShapeDtypeStruct((w_iter * wrows,), F32),
        mesh=mesh,
        scratch_types=[
            pltpu.VMEM((NWINP + 16,), jnp.int32),
            pltpu.VMEM((_ACC_CHUNK * MW,), F32),
            pltpu.VMEM((_ACC_CHUNK * MW,), F32),
            pltpu.VMEM((wrows,), F32),
            pltpu.SemaphoreType.DMA,
            pltpu.SemaphoreType.DMA,
            pltpu.SemaphoreType.DMA,
        ],
    )
    def k(g_hbm, st_hbm, out_hbm, starts_v, chunk_v, chunk2_v, win_v, sem,
          semb, semo):
        wid = lax.axis_index("s") * NC + lax.axis_index("c")
        pltpu.sync_copy(st_hbm, starts_v)
        trips = (w_iter - wid + NW - 1) // NW

        def wloop(t, c0):
            w = wid + t * NW
            sv = starts_v[pl.ds(w, 16)]
            s0 = sv[0]
            cnt = sv[1] - s0

            def zloop(z, c1):
                win_v[pl.ds(z * 16, 16)] = jnp.zeros((16,), F32)
                return c1

            lax.fori_loop(0, wrows // 16, zloop, 0)

            nchk = (cnt + _ACC_CHUNK - 1) // _ACC_CHUNK

            def start_read(ch, cb, sm):
                pltpu.async_copy(
                    g_hbm.at[pl.ds((s0 + ch * _ACC_CHUNK) * MW, _ACC_CHUNK * MW)],
                    cb,
                    sm,
                )

            def wait_read(cb, sm):
                pltpu.make_async_copy(
                    g_hbm.at[pl.ds(0, _ACC_CHUNK * MW)], cb, sm
                ).wait()

            @pl.when(nchk > 0)
            def _():
                start_read(0, chunk_v, sem)

            def chunk_body(ch, cbuf, obuf, sm, osm):
                wait_read(cbuf, sm)

                @pl.when(ch + 1 < nchk)
                def _():
                    start_read(ch + 1, obuf, osm)

                nrows = jnp.minimum(jnp.int32(_ACC_CHUNK), cnt - ch * _ACC_CHUNK)

                def row_loop(j, c3):
                    dl = cbuf[pl.ds(j * MW + 256, 16)][0].astype(jnp.int32)

                    def kloop(kk, c4):
                        off = dl * 256 + kk * 16
                        win_v[pl.ds(off, 16)] = (
                            win_v[pl.ds(off, 16)] + cbuf[pl.ds(j * MW + kk * 16, 16)]
                        )
                        return c4

                    lax.fori_loop(0, 16, kloop, 0)
                    return c3

                lax.fori_loop(0, nrows, row_loop, 0)

            def cpair(i, c2):
                @pl.when(2 * i < nchk)
                def _():
                    chunk_body(2 * i, chunk_v, chunk2_v, sem, semb)

                @pl.when(2 * i + 1 < nchk)
                def _():
                    chunk_body(2 * i + 1, chunk2_v, chunk_v, semb, sem)

                return c2

            lax.fori_loop(0, (nchk + 1) // 2, cpair, 0)

            def oloop(sub, c5):
                pltpu.sync_copy(
                    win_v.at[pl.ds(sub * 4096, 4096)],
                    out_hbm.at[pl.ds(w * wrows + sub * 4096, 4096)],
                )
                return c5

            lax.fori_loop(0, wrows // 4096, oloop, 0)
            return c0

        lax.fori_loop(0, trips, wloop, 0)

    return k(grouped_flat, starts)


def _sc_scatter_add(msgs, dst, n_out):
    nwin = -(-n_out // WPC)
    w_iter = -(-nwin // NW) * NW
    _, starts = _win_count(dst, 2000)
    slot = _slot(dst, starts, 640)
    grouped = _sc_regroup(msgs, slot.reshape(-1))
    out_flat = _sc_win_accum(
        grouped.reshape(-1), starts[0].astype(jnp.int32), w_iter
    )
    return out_flat.reshape(w_iter * WPC, 256)[:n_out]


def _qcconv(c, h, src, dst, efeat, n_nodes, P, rows_blk):
    wn = jnp.concatenate(
        [P["Wkv"][c][0], P["Wvv"][c][0], P["Wkv"][c][1], P["Wvv"][c][1]], axis=1
    )
    we4 = jnp.concatenate(
        [P["Wke"][c][0], P["Wve"][c][0], P["Wke"][c][1], P["Wve"][c][1]], axis=1
    )
    t1, t2 = _proj(h, wn, rows_blk)
    te = _mm(efeat, we4, 2000)
    gsrc = _sc_gather(t1, src)
    gdst = _sc_gather(t2, dst)
    msgs = _edge_mlp(
        gsrc, gdst, te, dst,
        P["Wu"][c][0], P["Wu"][c][1], P["bu"][c][0], P["bu"][c][1],
        P["Wm"][c][0], P["Wm"][c][1], P["bm"][c][0], P["bm"][c][1],
        1000,
    )
    hv = _sc_scatter_add(msgs, dst, n_nodes)
    y, stats = _cat_stats(hv, P["Wcat"][c], P["bcat"][c], rows_blk)
    return _bn_res(y, stats, h, rows_blk)


def kernel(x, edge_index, edge_dis, triangle_index, triangle_dis, x_batch,
           edge_dis_batch, Wkv, Wke, Wvv, Wve, Wu, bu, Wm, bm, Wcat, bcat, Wa,
           ba, We, be, Wt, bt, Wfc, bfc, Wfc2, bfc2, Wo, bo):
    P = {"Wkv": Wkv, "Wke": Wke, "Wvv": Wvv, "Wve": Wve, "Wu": Wu, "bu": bu,
         "Wm": Wm, "bm": bm, "Wcat": Wcat, "bcat": bcat}
    src_t = triangle_index[0]
    dst_t = triangle_index[1]
    src_e = edge_index[0]
    dst_e = edge_index[1]

    h0 = _node_embed(x, Wa, ba, 1000)
    ef0 = _feat_embed(edge_dis, We, be, 64, 2000)
    tf = _feat_embed(triangle_dis, Wt, bt, 80, 2000)

    ef1 = _qcconv(0, ef0, src_t, dst_t, tf, E, P, 2000)
    h1 = _qcconv(1, h0, src_e, dst_e, ef1, N, P, 1000)

    s1, c1 = _pool(h1, x_batch, 1000)
    s2, c2 = _pool(ef1, edge_dis_batch, 2000)
    return _head(s1, c1, s2, c2, Wfc, bfc, Wfc2, bfc2, Wo, bo)


# speedup vs baseline: 1.1661x; 1.1021x over previous
import functools

import jax
import jax.numpy as jnp
from jax import lax
from jax.experimental import pallas as pl
from jax.experimental.pallas import tpu as pltpu
from jax.experimental.pallas import tpu_sc as plsc

N = 10000
E = 160000
D = 128

NC = 2
NS = 16
NW = NC * NS

WPC = 256
NWINP = 768
MW = 384

F32 = jnp.float32


def _leaky(v):
    return jnp.where(v >= 0, v, 0.01 * v)


def _node_embed_body(x_ref, w_ref, b_ref, o_ref):
    o_ref[...] = (
        jnp.dot(x_ref[...], w_ref[...], preferred_element_type=F32) + b_ref[...]
    )


def _node_embed(x, w, b, rows_blk):
    n = x.shape[0]
    grid = n // rows_blk
    return pl.pallas_call(
        _node_embed_body,
        grid=(grid,),
        in_specs=[
            pl.BlockSpec((rows_blk, x.shape[1]), lambda i: (i, 0)),
            pl.BlockSpec(w.shape, lambda i: (0, 0)),
            pl.BlockSpec((1, D), lambda i: (0, 0)),
        ],
        out_specs=pl.BlockSpec((rows_blk, D), lambda i: (i, 0)),
        out_shape=jax.ShapeDtypeStruct((n, D), F32),
    )(x, w, b.reshape(1, D))


def _feat_embed_body(d_ref, c_ref, w_ref, b_ref, o_ref, *, bins, gamma):
    d = d_ref[...]
    c = c_ref[...]
    w = w_ref[...]
    acc = jnp.zeros((d.shape[0], D), F32)
    for i in range(3):
        di = d[:, i : i + 1]
        r = jnp.exp(-gamma * (di - c) ** 2)
        acc += jnp.dot(r, w[i * bins : (i + 1) * bins, :], preferred_element_type=F32)
    acc += jnp.dot(d[:, 3:], w[3 * bins :, :], preferred_element_type=F32)
    o_ref[...] = acc + b_ref[...]


def _feat_embed(d, w, b, bins, rows_blk):
    n, f = d.shape
    gamma = 1.0 / (8.0 / bins) ** 2
    centers = jnp.linspace(0.0, 8.0, bins, dtype=F32).reshape(1, bins)
    grid = n // rows_blk
    body = functools.partial(_feat_embed_body, bins=bins, gamma=gamma)
    return pl.pallas_call(
        body,
        grid=(grid,),
        in_specs=[
            pl.BlockSpec((rows_blk, f), lambda i: (i, 0)),
            pl.BlockSpec((1, bins), lambda i: (0, 0)),
            pl.BlockSpec(w.shape, lambda i: (0, 0)),
            pl.BlockSpec((1, D), lambda i: (0, 0)),
        ],
        out_specs=pl.BlockSpec((rows_blk, D), lambda i: (i, 0)),
        out_shape=jax.ShapeDtypeStruct((n, D), F32),
    )(d, centers, w, b.reshape(1, D))


def _proj_body(h_ref, w_ref, o1_ref, o2_ref):
    t = jnp.dot(h_ref[...], w_ref[...], preferred_element_type=F32)
    o1_ref[...] = t
    o2_ref[...] = jnp.concatenate([t[:, 0:D], t[:, 2 * D : 3 * D]], axis=1)


def _proj(h, w4, rows_blk):
    n = h.shape[0]
    grid = n // rows_blk
    return pl.pallas_call(
        _proj_body,
        grid=(grid,),
        in_specs=[
            pl.BlockSpec((rows_blk, D), lambda i: (i, 0)),
            pl.BlockSpec((D, 4 * D), lambda i: (0, 0)),
        ],
        out_specs=[
            pl.BlockSpec((rows_blk, 4 * D), lambda i: (i, 0)),
            pl.BlockSpec((rows_blk, 2 * D), lambda i: (i, 0)),
        ],
        out_shape=[
            jax.ShapeDtypeStruct((n, 4 * D), F32),
            jax.ShapeDtypeStruct((n, 2 * D), F32),
        ],
    )(h, w4)


def _mm_body(x_ref, w_ref, o_ref):
    o_ref[...] = jnp.dot(x_ref[...], w_ref[...], preferred_element_type=F32)


def _mm(x, w, rows_blk):
    n, k = x.shape
    m = w.shape[1]
    grid = n // rows_blk
    return pl.pallas_call(
        _mm_body,
        grid=(grid,),
        in_specs=[
            pl.BlockSpec((rows_blk, k), lambda i: (i, 0)),
            pl.BlockSpec((k, m), lambda i: (0, 0)),
        ],
        out_specs=pl.BlockSpec((rows_blk, m), lambda i: (i, 0)),
        out_shape=jax.ShapeDtypeStruct((n, m), F32),
    )(x, w)


def _edge_mlp_body(
    gs_ref, gd_ref, te_ref, dst_ref, wu0_ref, wu1_ref, bu0_ref, bu1_ref,
    wm0_ref, wm1_ref, bm0_ref, bm1_ref, o_ref,
):
    gs = gs_ref[...]
    gd = gd_ref[...]
    te = te_ref[...]
    wu_refs = (wu0_ref, wu1_ref)
    bu_refs = (bu0_ref, bu1_ref)
    wm_refs = (wm0_ref, wm1_ref)
    bm_refs = (bm0_ref, bm1_ref)
    for hd in range(2):
        q = gd[:, hd * D : (hd + 1) * D]
        k = gs[:, 2 * hd * D : (2 * hd + 1) * D]
        v = gs[:, (2 * hd + 1) * D : (2 * hd + 2) * D]
        ke = te[:, 2 * hd * D : (2 * hd + 1) * D]
        ve = te[:, (2 * hd + 1) * D : (2 * hd + 2) * D]
        a0 = q * k * (1.0 / 16.0)
        a1 = q * ke * (1.0 / 16.0)
        mu = (jnp.sum(a0, 1, keepdims=True) + jnp.sum(a1, 1, keepdims=True)) * (
            1.0 / (2.0 * D)
        )
        d0 = a0 - mu
        d1 = a1 - mu
        var = (jnp.sum(d0 * d0, 1, keepdims=True) + jnp.sum(d1 * d1, 1, keepdims=True)) * (
            1.0 / (2.0 * D)
        )
        sd = jnp.sqrt(var + 1e-5)
        g0 = jax.nn.sigmoid(d0 / sd)
        g1 = jax.nn.sigmoid(d1 / sd)
        wu = wu_refs[hd][...]
        u = (
            jnp.dot(v, wu[0:D, :], preferred_element_type=F32)
            + jnp.dot(ve, wu[D : 2 * D, :], preferred_element_type=F32)
            + bu_refs[hd][...]
        )
        wm = wm_refs[hd][...]
        y = (
            jnp.dot(u[:, 0:D] * g0, wm[0:D, :], preferred_element_type=F32)
            + jnp.dot(u[:, D : 2 * D] * g1, wm[D : 2 * D, :], preferred_element_type=F32)
            + bm_refs[hd][...]
        )
        mu2 = jnp.sum(y, 1, keepdims=True) * (1.0 / D)
        dy = y - mu2
        var2 = jnp.sum(dy * dy, 1, keepdims=True) * (1.0 / D)
        yn = dy / jnp.sqrt(var2 + 1e-5)
        o_ref[:, hd * D : (hd + 1) * D] = _leaky(yn)
    dst = dst_ref[...]
    dl = (dst - (dst // WPC) * WPC).astype(F32)
    o_ref[:, 2 * D : 2 * D + D] = dl * jnp.ones((1, D), F32)


def _edge_mlp(gsrc, gdst, te, dst, wu0, wu1, bu0, bu1, wm0, wm1, bm0, bm1,
              rows_blk):
    n = gsrc.shape[0]
    grid = n // rows_blk
    wspec = lambda shape: pl.BlockSpec(shape, lambda i: (0, 0))
    return pl.pallas_call(
        _edge_mlp_body,
        grid=(grid,),
        in_specs=[
            pl.BlockSpec((rows_blk, 4 * D), lambda i: (i, 0)),
            pl.BlockSpec((rows_blk, 2 * D), lambda i: (i, 0)),
            pl.BlockSpec((rows_blk, 4 * D), lambda i: (i, 0)),
            pl.BlockSpec((rows_blk, 1), lambda i: (i, 0)),
            wspec((2 * D, 2 * D)),
            wspec((2 * D, 2 * D)),
            wspec((1, 2 * D)),
            wspec((1, 2 * D)),
            wspec((2 * D, D)),
            wspec((2 * D, D)),
            wspec((1, D)),
            wspec((1, D)),
        ],
        out_specs=pl.BlockSpec((rows_blk, MW), lambda i: (i, 0)),
        out_shape=jax.ShapeDtypeStruct((n, MW), F32),
    )(
        gsrc, gdst, te, dst.reshape(n, 1), wu0, wu1,
        bu0.reshape(1, 2 * D), bu1.reshape(1, 2 * D),
        wm0, wm1, bm0.reshape(1, D), bm1.reshape(1, D),
    )


def _win_count_body(dst_ref, cnt_ref, st_ref):
    @pl.when(pl.program_id(0) == 0)
    def _():
        cnt_ref[...] = jnp.zeros_like(cnt_ref)
        st_ref[...] = jnp.zeros_like(st_ref)

    win = dst_ref[...] // WPC
    seg = lax.broadcasted_iota(jnp.int32, (1, NWINP), 1)
    oh = (win == seg).astype(F32)
    cnt_ref[0:1, :] += jnp.sum(oh, 0, keepdims=True)

    @pl.when(pl.program_id(0) == pl.num_programs(0) - 1)
    def _():
        counts = cnt_ref[0:1, :]
        i = lax.broadcasted_iota(jnp.int32, (NWINP, NWINP + 16), 0)
        j = lax.broadcasted_iota(jnp.int32, (NWINP, NWINP + 16), 1)
        m = (i < j).astype(F32)
        base = jnp.dot(counts, m, preferred_element_type=F32,
                       precision=lax.Precision.HIGHEST)
        st_ref[0:1, :] = base.astype(jnp.int32)


def _win_count(dst, rows_blk):
    n = dst.shape[0]
    grid = n // rows_blk
    return pl.pallas_call(
        _win_count_body,
        grid=(grid,),
        in_specs=[pl.BlockSpec((rows_blk, 1), lambda i: (i, 0))],
        out_specs=[
            pl.BlockSpec((8, NWINP), lambda i: (0, 0)),
            pl.BlockSpec((8, NWINP + 16), lambda i: (0, 0)),
        ],
        out_shape=[
            jax.ShapeDtypeStruct((8, NWINP), F32),
            jax.ShapeDtypeStruct((8, NWINP + 16), jnp.int32),
        ],
    )(dst.reshape(n, 1))


def _slot_body(dstc_ref, dstr_ref, st_ref, slot_ref, carry_ref):
    @pl.when(pl.program_id(0) == 0)
    def _():
        carry_ref[...] = jnp.zeros_like(carry_ref)

    b = dstc_ref.shape[0]
    win = jnp.floor(dstc_ref[...] * (1.0 / WPC))
    win_r = jnp.floor(dstr_ref[...] * (1.0 / WPC))
    seg = lax.broadcasted_iota(jnp.int32, (1, NWINP), 1).astype(F32)
    oh = (win == seg).astype(F32)
    eq = (win == win_r)
    qi = lax.broadcasted_iota(jnp.int32, (b, b), 1)
    pi = lax.broadcasted_iota(jnp.int32, (b, b), 0)
    low = qi < pi
    rank = jnp.sum((eq & low).astype(F32), axis=1, keepdims=True)
    base = st_ref[0:1, 0:NWINP].astype(F32) + carry_ref[0:1, :]
    dn = (((1,), (1,)), ((), ()))
    off = lax.dot_general(oh, base, dn, preferred_element_type=F32,
                          precision=lax.Precision.HIGHEST)
    slot_ref[...] = (off + rank).astype(jnp.int32)
    carry_ref[0:1, :] += jnp.sum(oh, 0, keepdims=True)


def _slot(dst, starts, rows_blk):
    n = dst.shape[0]
    grid = n // rows_blk
    dstf = dst.astype(F32)
    slot, _ = pl.pallas_call(
        _slot_body,
        grid=(grid,),
        in_specs=[
            pl.BlockSpec((rows_blk, 1), lambda i: (i, 0)),
            pl.BlockSpec((1, rows_blk), lambda i: (0, i)),
            pl.BlockSpec((8, NWINP + 16), lambda i: (0, 0)),
        ],
        out_specs=[
            pl.BlockSpec((rows_blk, 1), lambda i: (i, 0)),
            pl.BlockSpec((8, NWINP), lambda i: (0, 0)),
        ],
        out_shape=[
            jax.ShapeDtypeStruct((n, 1), jnp.int32),
            jax.ShapeDtypeStruct((8, NWINP), F32),
        ],
    )(dstf.reshape(n, 1), dstf.reshape(1, n), starts)
    return slot


def _cat_stats_body(hv_ref, w_ref, b_ref, y_ref, st_ref):
    y = jnp.dot(hv_ref[...], w_ref[...], preferred_element_type=F32) + b_ref[...]
    y_ref[...] = y

    @pl.when(pl.program_id(0) == 0)
    def _():
        st_ref[...] = jnp.zeros_like(st_ref)

    st_ref[0:1, :] += jnp.sum(y, 0, keepdims=True)
    st_ref[1:2, :] += jnp.sum(y * y, 0, keepdims=True)


def _cat_stats(hv, wcat, bcat, rows_blk):
    n = hv.shape[0]
    grid = n // rows_blk
    return pl.pallas_call(
        _cat_stats_body,
        grid=(grid,),
        in_specs=[
            pl.BlockSpec((rows_blk, 2 * D), lambda i: (i, 0)),
            pl.BlockSpec((2 * D, D), lambda i: (0, 0)),
            pl.BlockSpec((1, D), lambda i: (0, 0)),
        ],
        out_specs=[
            pl.BlockSpec((rows_blk, D), lambda i: (i, 0)),
            pl.BlockSpec((8, D), lambda i: (0, 0)),
        ],
        out_shape=[
            jax.ShapeDtypeStruct((n, D), F32),
            jax.ShapeDtypeStruct((8, D), F32),
        ],
    )(hv, wcat, bcat.reshape(1, D))


def _bn_res_body(y_ref, st_ref, h_ref, o_ref, *, n):
    st = st_ref[...]
    m = st[0:1, :] * (1.0 / n)
    var = st[1:2, :] * (1.0 / n) - m * m
    o_ref[...] = _leaky((y_ref[...] - m) / jnp.sqrt(var + 1e-5)) + h_ref[...]


def _bn_res(y, stats, h, rows_blk):
    n = y.shape[0]
    grid = n // rows_blk
    body = functools.partial(_bn_res_body, n=float(n))
    return pl.pallas_call(
        body,
        grid=(grid,),
        in_specs=[
            pl.BlockSpec((rows_blk, D), lambda i: (i, 0)),
            pl.BlockSpec((8, D), lambda i: (0, 0)),
            pl.BlockSpec((rows_blk, D), lambda i: (i, 0)),
        ],
        out_specs=pl.BlockSpec((rows_blk, D), lambda i: (i, 0)),
        out_shape=jax.ShapeDtypeStruct((n, D), F32),
    )(y, stats, h)


def _pool_body(h_ref, b_ref, s_ref, c_ref):
    @pl.when(pl.program_id(0) == 0)
    def _():
        s_ref[...] = jnp.zeros_like(s_ref)
        c_ref[...] = jnp.zeros_like(c_ref)

    h = h_ref[...]
    bids = b_ref[...]
    seg = lax.broadcasted_iota(jnp.int32, (1, 64), 1)
    oh = (bids == seg).astype(F32)
    dn = (((0,), (0,)), ((), ()))
    s_ref[...] += lax.dot_general(oh, h, dn, preferred_element_type=F32,
                                  precision=lax.Precision.HIGHEST)
    c_ref[...] += lax.dot_general(oh, jnp.ones_like(h), dn, preferred_element_type=F32,
                                  precision=lax.Precision.HIGHEST)


def _pool(h, bids, rows_blk):
    n = h.shape[0]
    grid = n // rows_blk
    return pl.pallas_call(
        _pool_body,
        grid=(grid,),
        in_specs=[
            pl.BlockSpec((rows_blk, D), lambda i: (i, 0)),
            pl.BlockSpec((rows_blk, 1), lambda i: (i, 0)),
        ],
        out_specs=[
            pl.BlockSpec((64, D), lambda i: (0, 0)),
            pl.BlockSpec((64, D), lambda i: (0, 0)),
        ],
        out_shape=[
            jax.ShapeDtypeStruct((64, D), F32),
            jax.ShapeDtypeStruct((64, D), F32),
        ],
    )(h, bids.reshape(n, 1))


def _head_body(s1_ref, c1_ref, s2_ref, c2_ref, wfc_ref, bfc_ref, wfc2_ref,
               bfc2_ref, wo_ref, bo_ref, o_ref):
    f1 = s1_ref[...] / jnp.maximum(c1_ref[...], 1.0)
    f2 = s2_ref[...] / jnp.maximum(c2_ref[...], 1.0)
    wfc = wfc_ref[...]
    a = (
        jnp.dot(f1, wfc[0:D, :], preferred_element_type=F32)
        + jnp.dot(f2, wfc[D : 2 * D, :], preferred_element_type=F32)
        + bfc_ref[...]
    )
    a = _leaky(a)
    a = _leaky(jnp.dot(a, wfc2_ref[...], preferred_element_type=F32) + bfc2_ref[...])
    o_ref[...] = jnp.dot(a, wo_ref[...], preferred_element_type=F32) + bo_ref[...]


def _head(s1, c1, s2, c2, wfc, bfc, wfc2, bfc2, wo, bo):
    wo_pad = jnp.zeros((D, D), F32).at[:, 0].set(wo[:, 0])
    bo_pad = jnp.zeros((1, D), F32).at[0, 0].set(bo[0])
    full = lambda a: pl.BlockSpec(a.shape, lambda: (0,) * a.ndim)
    args = (s1, c1, s2, c2, wfc, bfc.reshape(1, D), wfc2, bfc2.reshape(1, D),
            wo_pad, bo_pad)
    out = pl.pallas_call(
        _head_body,
        in_specs=[full(a) for a in args],
        out_specs=pl.BlockSpec((64, D), lambda: (0, 0)),
        out_shape=jax.ShapeDtypeStruct((64, D), F32),
    )(*args)
    return out[:, 0]


_GATHER_CHUNK = 128
_ACC_CHUNK = 64


_GC = 40


def _sc_gather(table, idx):
    n, width = table.shape
    e = idx.shape[0]
    per_w = e // NW
    nch = per_w // _GC
    mesh = plsc.VectorSubcoreMesh(
        core_axis_name="c", subcore_axis_name="s", num_cores=NC, num_subcores=NS
    )

    nb_ = 4

    @functools.partial(
        pl.kernel,
        out_type=jax.ShapeDtypeStruct((e, width), F32),
        mesh=mesh,
        scratch_types=(
            [pltpu.VMEM((per_w,), jnp.int32)]
            + [pltpu.VMEM((_GC, width), F32) for _ in range(4)]
            + [pltpu.SemaphoreType.DMA for _ in range(8)]
        ),
    )
    def k(tab_hbm, idx_hbm, out_hbm, idx_v, r0, r1, r2, r3,
          sg0, sg1, sg2, sg3, sw0, sw1, sw2, sw3):
        wid = lax.axis_index("s") * NC + lax.axis_index("c")
        w0 = wid * per_w
        pltpu.sync_copy(idx_hbm.at[pl.ds(w0, per_w)], idx_v)
        bufs = (r0, r1, r2, r3)
        sgs = (sg0, sg1, sg2, sg3)
        sws = (sw0, sw1, sw2, sw3)

        def start_gather(c, b):
            pltpu.async_copy(
                tab_hbm.at[idx_v.at[pl.ds(c * _GC, _GC)]], bufs[b], sgs[b]
            )

        def wait_gather(b):
            pltpu.make_async_copy(tab_hbm.at[idx_v.at[pl.ds(0, _GC)]], bufs[b],
                                  sgs[b]).wait()

        def start_write(c, b):
            pltpu.async_copy(bufs[b], out_hbm.at[pl.ds(w0 + c * _GC, _GC)], sws[b])

        def wait_write(b):
            pltpu.make_async_copy(bufs[b], out_hbm.at[pl.ds(w0, _GC)], sws[b]).wait()

        start_gather(0, 0)
        start_gather(1, 1)

        def ring(i, carry):
            for b in range(nb_):
                c = i * nb_ + b
                nxt = (b + 2) % nb_

                @pl.when(c + 2 < nch)
                def _():
                    @pl.when(c >= 2)
                    def _():
                        wait_write(nxt)

                    start_gather(c + 2, nxt)

                wait_gather(b)
                start_write(c, b)
            return carry

        lax.fori_loop(0, nch // nb_, ring, 0)
        for c in range(nch - nch % nb_, nch):
            b = c % nb_
            wait_gather(b)
            start_write(c, b)
        for b in range(nb_):
            wait_write(b)

    return k(table, idx)


def _sc_regroup(msgs, slot):
    e = msgs.shape[0]
    per_w = e // NW
    nch = per_w // _GC
    mesh = plsc.VectorSubcoreMesh(
        core_axis_name="c", subcore_axis_name="s", num_cores=NC, num_subcores=NS
    )

    @functools.partial(
        pl.kernel,
        out_type=jax.ShapeDtypeStruct((e + _ACC_CHUNK, MW), F32),
        mesh=mesh,
        scratch_types=[
            pltpu.VMEM((per_w,), jnp.int32),
            pltpu.VMEM((_GC, MW), F32),
            pltpu.VMEM((_GC, MW), F32),
            pltpu.SemaphoreType.DMA,
            pltpu.SemaphoreType.DMA,
            pltpu.SemaphoreType.DMA,
            pltpu.SemaphoreType.DMA,
        ],
    )
    def k(msgs_hbm, slot_hbm, out_hbm, idx_v, r0, r1, sg0, sg1, sw0, sw1):
        wid = lax.axis_index("s") * NC + lax.axis_index("c")
        w0 = wid * per_w
        pltpu.sync_copy(slot_hbm.at[pl.ds(w0, per_w)], idx_v)
        bufs = (r0, r1)
        sgs = (sg0, sg1)
        sws = (sw0, sw1)

        def start_read(c, b):
            pltpu.async_copy(msgs_hbm.at[pl.ds(w0 + c * _GC, _GC)], bufs[b], sgs[b])

        def wait_read(b):
            pltpu.make_async_copy(msgs_hbm.at[pl.ds(w0, _GC)], bufs[b], sgs[b]).wait()

        def start_write(c, b):
            pltpu.async_copy(
                bufs[b], out_hbm.at[idx_v.at[pl.ds(c * _GC, _GC)]], sws[b]
            )

        def wait_write(b):
            pltpu.make_async_copy(
                bufs[b], out_hbm.at[idx_v.at[pl.ds(0, _GC)]], sws[b]
            ).wait()

        start_read(0, 0)

        def pair(i, carry):
            for b in range(2):
                c = i * 2 + b
                nb = 1 - b

                @pl.when(c + 1 < nch)
                def _():
                    @pl.when(c >= 1)
                    def _():
                        wait_write(nb)

                    start_read(c + 1, nb)

                wait_read(b)
                start_write(c, b)
            return carry

        lax.fori_loop(0, nch // 2, pair, 0)
        if nch % 2:
            c = nch - 1
            wait_read(c % 2)
            start_write(c, c % 2)
        wait_write(0)
        wait_write(1)

    return k(msgs, slot)


def _sc_win_accum(grouped_flat, starts, w_iter):
    mesh = plsc.VectorSubcoreMesh(
        core_axis_name="c", subcore_axis_name="s", num_cores=NC, num_subcores=NS
    )
    wrows = WPC * 256

    @functools.partial(
        pl.kernel,
        out_type=jax.ShapeDtypeStruct((w_iter * wrows,), F32),
        mesh=mesh,
        scratch_types=[
            pltpu.VMEM((NWINP + 16,), jnp.int32),
            pltpu.VMEM((_ACC_CHUNK * MW,), F32),
            pltpu.VMEM((_ACC_CHUNK * MW,), F32),
            pltpu.VMEM((wrows,), F32),
            pltpu.SemaphoreType.DMA,
            pltpu.SemaphoreType.DMA,
            pltpu.SemaphoreType.DMA,
        ],
    )
    def k(g_hbm, st_hbm, out_hbm, starts_v, chunk_v, chunk2_v, win_v, sem,
          semb, semo):
        wid = lax.axis_index("s") * NC + lax.axis_index("c")
        pltpu.sync_copy(st_hbm, starts_v)
        trips = (w_iter - wid + NW - 1) // NW

        def wloop(t, c0):
            w = wid + t * NW
            sv = starts_v[pl.ds(w, 16)]
            s0 = sv[0]
            cnt = sv[1] - s0

            zv = jnp.zeros((16,), F32)

            def zloop(z, c1):
                for zz in range(16):
                    win_v[pl.ds(z * 256 + zz * 16, 16)] = zv
                return c1

            lax.fori_loop(0, wrows // 256, zloop, 0)

            nchk = (cnt + _ACC_CHUNK - 1) // _ACC_CHUNK

            def start_read(ch, cb, sm):
                pltpu.async_copy(
                    g_hbm.at[pl.ds((s0 + ch * _ACC_CHUNK) * MW, _ACC_CHUNK * MW)],
                    cb,
                    sm,
                )

            def wait_read(cb, sm):
                pltpu.make_async_copy(
                    g_hbm.at[pl.ds(0, _ACC_CHUNK * MW)], cb, sm
                ).wait()

            @pl.when(nchk > 0)
            def _():
                start_read(0, chunk_v, sem)

            def chunk_body(ch, cbuf, obuf, sm, osm):
                wait_read(cbuf, sm)

                @pl.when(ch + 1 < nchk)
                def _():
                    start_read(ch + 1, obuf, osm)

                nrows = jnp.minimum(jnp.int32(_ACC_CHUNK), cnt - ch * _ACC_CHUNK)

                def row_loop(j, c3):
                    dl = cbuf[pl.ds(j * MW + 256, 16)][0].astype(jnp.int32)
                    wbase = dl * 256
                    rbase = j * MW
                    for kk in range(16):
                        win_v[pl.ds(wbase + kk * 16, 16)] = (
                            win_v[pl.ds(wbase + kk * 16, 16)]
                            + cbuf[pl.ds(rbase + kk * 16, 16)]
                        )
                    return c3

                lax.fori_loop(0, nrows, row_loop, 0)

            def cpair(i, c2):
                @pl.when(2 * i < nchk)
                def _():
                    chunk_body(2 * i, chunk_v, chunk2_v, sem, semb)

                @pl.when(2 * i + 1 < nchk)
                def _():
                    chunk_body(2 * i + 1, chunk2_v, chunk_v, semb, sem)

                return c2

            lax.fori_loop(0, (nchk + 1) // 2, cpair, 0)

            def oloop(sub, c5):
                pltpu.sync_copy(
                    win_v.at[pl.ds(sub * 4096, 4096)],
                    out_hbm.at[pl.ds(w * wrows + sub * 4096, 4096)],
                )
                return c5

            lax.fori_loop(0, wrows // 4096, oloop, 0)
            return c0

        lax.fori_loop(0, trips, wloop, 0)

    return k(grouped_flat, starts)


def _sc_scatter_add(msgs, dst, n_out):
    nwin = -(-n_out // WPC)
    w_iter = -(-nwin // NW) * NW
    _, starts = _win_count(dst, 2000)
    slot = _slot(dst, starts, 640)
    grouped = _sc_regroup(msgs, slot.reshape(-1))
    out_flat = _sc_win_accum(
        grouped.reshape(-1), starts[0].astype(jnp.int32), w_iter
    )
    return out_flat.reshape(w_iter * WPC, 256)[:n_out]


def _qcconv(c, h, src, dst, efeat, n_nodes, P, rows_blk):
    wn = jnp.concatenate(
        [P["Wkv"][c][0], P["Wvv"][c][0], P["Wkv"][c][1], P["Wvv"][c][1]], axis=1
    )
    we4 = jnp.concatenate(
        [P["Wke"][c][0], P["Wve"][c][0], P["Wke"][c][1], P["Wve"][c][1]], axis=1
    )
    t1, t2 = _proj(h, wn, rows_blk)
    te = _mm(efeat, we4, 2000)
    gsrc = _sc_gather(t1, src)
    gdst = _sc_gather(t2, dst)
    msgs = _edge_mlp(
        gsrc, gdst, te, dst,
        P["Wu"][c][0], P["Wu"][c][1], P["bu"][c][0], P["bu"][c][1],
        P["Wm"][c][0], P["Wm"][c][1], P["bm"][c][0], P["bm"][c][1],
        1000,
    )
    hv = _sc_scatter_add(msgs, dst, n_nodes)
    y, stats = _cat_stats(hv, P["Wcat"][c], P["bcat"][c], rows_blk)
    return _bn_res(y, stats, h, rows_blk)


def kernel(x, edge_index, edge_dis, triangle_index, triangle_dis, x_batch,
           edge_dis_batch, Wkv, Wke, Wvv, Wve, Wu, bu, Wm, bm, Wcat, bcat, Wa,
           ba, We, be, Wt, bt, Wfc, bfc, Wfc2, bfc2, Wo, bo):
    P = {"Wkv": Wkv, "Wke": Wke, "Wvv": Wvv, "Wve": Wve, "Wu": Wu, "bu": bu,
         "Wm": Wm, "bm": bm, "Wcat": Wcat, "bcat": bcat}
    src_t = triangle_index[0]
    dst_t = triangle_index[1]
    src_e = edge_index[0]
    dst_e = edge_index[1]

    h0 = _node_embed(x, Wa, ba, 1000)
    ef0 = _feat_embed(edge_dis, We, be, 64, 2000)
    tf = _feat_embed(triangle_dis, Wt, bt, 80, 2000)

    ef1 = _qcconv(0, ef0, src_t, dst_t, tf, E, P, 2000)
    h1 = _qcconv(1, h0, src_e, dst_e, ef1, N, P, 1000)

    s1, c1 = _pool(h1, x_batch, 1000)
    s2, c2 = _pool(ef1, edge_dis_batch, 2000)
    return _head(s1, c1, s2, c2, Wfc, bfc, Wfc2, bfc2, Wo, bo)
